# baseline (speedup 1.0000x reference)
import jax
import jax.numpy as jnp
from jax.experimental import pallas as pl
from jax.experimental.pallas import tpu as pltpu

_ROW_CHUNKS = 4
_COL_CHUNKS = 2
_N_CHUNKS = _ROW_CHUNKS * _COL_CHUNKS


def _staged_copy(table_hbm, out_hbm, buf, sem_in, sem_out):
    rows, cols = table_hbm.shape
    rc, cc = rows // _ROW_CHUNKS, cols // _COL_CHUNKS

    def idx(i):
        r, c = divmod(i, _COL_CHUNKS)
        return (pl.ds(r * rc, rc), pl.ds(c * cc, cc))

    def cin(i):
        return pltpu.make_async_copy(
            table_hbm.at[idx(i)], buf.at[idx(i)], sem_in.at[i]
        )

    def cout(i):
        return pltpu.make_async_copy(
            buf.at[idx(i)], out_hbm.at[idx(i)], sem_out.at[i]
        )

    for i in range(_N_CHUNKS):
        cin(i).start()
    for i in range(_N_CHUNKS):
        cin(i).wait()
        cout(i).start()
    for i in range(_N_CHUNKS):
        cout(i).wait()


def kernel(input_indices, position_embedding_table):
    seq_len = input_indices.shape[1]
    n_rows, dim = position_embedding_table.shape
    return pl.pallas_call(
        _staged_copy,
        in_specs=[pl.BlockSpec(memory_space=pltpu.HBM)],
        out_specs=pl.BlockSpec(memory_space=pltpu.HBM),
        out_shape=jax.ShapeDtypeStruct((seq_len, dim), position_embedding_table.dtype),
        scratch_shapes=[
            pltpu.VMEM((seq_len, dim), position_embedding_table.dtype),
            pltpu.SemaphoreType.DMA((_N_CHUNKS,)),
            pltpu.SemaphoreType.DMA((_N_CHUNKS,)),
        ],
    )(position_embedding_table)


# speedup vs baseline: 1.0340x; 1.0340x over previous
import jax
import jax.numpy as jnp
from jax.experimental import pallas as pl
from jax.experimental.pallas import tpu as pltpu

_N_CHUNKS = 4


def _staged_copy(table_hbm, out_hbm, buf, sem_in, sem_out):
    rows = table_hbm.shape[0]
    chunk = rows // _N_CHUNKS

    def cin(i):
        return pltpu.make_async_copy(
            table_hbm.at[pl.ds(i * chunk, chunk)],
            buf.at[pl.ds(i * chunk, chunk)],
            sem_in.at[i],
        )

    def cout(i):
        return pltpu.make_async_copy(
            buf.at[pl.ds(i * chunk, chunk)],
            out_hbm.at[pl.ds(i * chunk, chunk)],
            sem_out.at[i],
        )

    for i in range(_N_CHUNKS):
        cin(i).start()
    for i in range(_N_CHUNKS):
        cin(i).wait()
        cout(i).start()
    for i in range(_N_CHUNKS):
        cout(i).wait()


def kernel(input_indices, position_embedding_table):
    seq_len = input_indices.shape[1]
    n_rows, dim = position_embedding_table.shape
    return pl.pallas_call(
        _staged_copy,
        in_specs=[pl.BlockSpec(memory_space=pltpu.HBM)],
        out_specs=pl.BlockSpec(memory_space=pltpu.HBM),
        out_shape=jax.ShapeDtypeStruct((seq_len, dim), position_embedding_table.dtype),
        scratch_shapes=[
            pltpu.VMEM((seq_len, dim), position_embedding_table.dtype),
            pltpu.SemaphoreType.DMA((_N_CHUNKS,)),
            pltpu.SemaphoreType.DMA((_N_CHUNKS,)),
        ],
    )(position_embedding_table)


# speedup vs baseline: 1.0514x; 1.0169x over previous
import jax
from jax.experimental import pallas as pl
from jax.experimental.pallas import tpu as pltpu

_N_CHUNKS = 4


def _staged_copy(table_hbm, out_hbm, buf, sem_in, sem_out):
    rows = out_hbm.shape[0]
    chunk = rows // _N_CHUNKS

    def cin(i):
        return pltpu.make_async_copy(
            table_hbm.at[pl.ds(i * chunk, chunk)],
            buf.at[pl.ds(i * chunk, chunk)],
            sem_in.at[i],
        )

    def cout(i):
        return pltpu.make_async_copy(
            buf.at[pl.ds(i * chunk, chunk)],
            out_hbm.at[pl.ds(i * chunk, chunk)],
            sem_out.at[i],
        )

    for i in range(_N_CHUNKS):
        cin(i).start()
    for i in range(_N_CHUNKS):
        cin(i).wait()
        cout(i).start()
    for i in range(_N_CHUNKS):
        cout(i).wait()


def kernel(input_indices, position_embedding_table):
    seq_len = input_indices.shape[1]
    n_rows, dim = position_embedding_table.shape
    return pl.pallas_call(
        _staged_copy,
        in_specs=[pl.BlockSpec(memory_space=pltpu.HBM)],
        out_specs=pl.BlockSpec(memory_space=pltpu.HBM),
        out_shape=jax.ShapeDtypeStruct((seq_len, dim), position_embedding_table.dtype),
        scratch_shapes=[
            pltpu.VMEM((seq_len, dim), position_embedding_table.dtype),
            pltpu.SemaphoreType.DMA((_N_CHUNKS,)),
            pltpu.SemaphoreType.DMA((_N_CHUNKS,)),
        ],
    )(position_embedding_table)
